# baseline (speedup 1.0000x reference)
import dataclasses
import functools

import jax
import jax.numpy as jnp
from jax import lax
from jax.experimental import pallas as pl
from jax.experimental.pallas import tpu as pltpu
from jax.experimental.pallas import tpu_sc as plsc

NC = 2
NS = 16
L = 16
NW = NC * NS


def _pc2_body(pos2_ref, col_ref, o_ref):
    for c0 in range(3):
        for c1 in range(3):
            o_ref[c0 * 3 + c1, :, 0:256] = (pos2_ref[:, 0, :]
                                            + col_ref[c0, :][None, :])
            o_ref[c0 * 3 + c1, :, 256:512] = (pos2_ref[:, 1, :]
                                              + col_ref[c1, :][None, :])


def _build_pc2_table(pos_t, color_table):
    T, D = pos_t.shape
    pos2 = pos_t.reshape(T // 2, 2, D)
    return pl.pallas_call(
        _pc2_body,
        out_shape=jax.ShapeDtypeStruct((9, T // 2, 2 * D), jnp.float32),
    )(pos2, color_table)


def _sc_gather_add(tok_packed, pc2_table, tok_idx, pc2_idx, W=32, K=5, A=3):
    N = tok_idx.shape[0]
    Dw = tok_packed.shape[1]
    D = 2 * Dw
    b_per_w = N // NW
    steps = b_per_w // W
    assert N % NW == 0 and b_per_w % W == 0 and W % 16 == 0
    assert steps % K == 0 and steps >= 2 * K and A < K
    mesh = plsc.VectorSubcoreMesh(core_axis_name="c", subcore_axis_name="s")

    scratch = (
        [pltpu.VMEM((b_per_w,), jnp.int32),
         pltpu.VMEM((b_per_w // 2,), jnp.int32)]
        + [pltpu.VMEM((W, Dw), jnp.int32)] * K
        + [pltpu.VMEM((W // 2, 2 * D), jnp.float32)] * K
        + [pltpu.SemaphoreType.DMA] * (2 * K)
    )

    cp = pltpu.CompilerParams()
    if "needs_layout_passes" in pltpu.CompilerParams.__dataclass_fields__:
        cp = dataclasses.replace(cp, needs_layout_passes=False)

    @functools.partial(
        pl.kernel,
        mesh=mesh,
        out_type=jax.ShapeDtypeStruct((N // 2, 2 * D), jnp.float32),
        scratch_types=scratch,
        compiler_params=cp,
    )
    def k(tok_tab, pc2_tab, tok_idx_hbm, pc2_idx_hbm, out_hbm, *sc):
        tok_i_v, pc2_i_v = sc[0], sc[1]
        ra = sc[2:2 + K]
        rb = sc[2 + K:2 + 2 * K]
        sg = sc[2 + 2 * K:2 + 3 * K]
        so = sc[2 + 3 * K:2 + 4 * K]
        wid = lax.axis_index("s") * NC + lax.axis_index("c")
        base_w = wid * b_per_w

        def _al(x):
            return x if isinstance(x, int) else pl.multiple_of(x, 8)

        pltpu.sync_copy(tok_idx_hbm.at[pl.ds(_al(base_w), b_per_w)], tok_i_v)
        pltpu.sync_copy(pc2_idx_hbm.at[pl.ds(_al(base_w // 2),
                                             b_per_w // 2)], pc2_i_v)

        def g_tok(i, p):
            off = _al(i * W)
            return pltpu.make_async_copy(
                tok_tab.at[tok_i_v.at[pl.ds(off, W)]], ra[p], sg[p])

        def g_pc(i, p):
            off = _al(i * (W // 2))
            return pltpu.make_async_copy(
                pc2_tab.at[pc2_i_v.at[pl.ds(off, W // 2)]], rb[p], sg[p])

        def out_cp(i, p):
            off = _al((base_w + i * W) // 2)
            return pltpu.make_async_copy(
                rb[p], out_hbm.at[pl.ds(off, W // 2)], so[p])

        def issue(i, p):
            g_tok(i, p).start()
            g_pc(i, p).start()

        def adds(p):
            @pl.loop(0, W // 2)
            def _pair(rp):
                for half in range(2):
                    for g in range(D // (2 * L)):
                        w = plsc.bitcast(
                            ra[p][2 * rp + half, pl.ds(L * g, L)],
                            jnp.bfloat16)
                        lo, hi = plsc.unpack(
                            w, format=plsc.PackFormat.INTERLEAVED)
                        base = half * D + 2 * L * g
                        plsc.addupdate(rb[p].at[rp, pl.ds(base, L)], lo)
                        plsc.addupdate(rb[p].at[rp, pl.ds(base + L, L)], hi)

        def consume(i, p):
            g_tok(i, p).wait()
            g_pc(i, p).wait()
            adds(p)
            out_cp(i, p).start()

        for i in range(A):
            issue(i, i % K)
        for i in range(K - A):
            issue(i + A, (i + A) % K)
            consume(i, i % K)

        @pl.loop(0, (steps - K) // K)
        def _grp(j):
            for p in range(K):
                i = (K - A) + j * K + p
                cbuf = (K - A + p) % K
                ibuf = (K - A + p + A) % K
                out_cp(i + A - K, ibuf).wait()
                issue(i + A, ibuf)
                consume(i, cbuf)

        for i in range(steps - A, steps):
            consume(i, i % K)
        for i in range(steps - K, steps):
            out_cp(i, i % K).wait()

    return k(tok_packed, pc2_table, tok_idx, pc2_idx)


def _swizzle_bf16(table):
    V, D = table.shape
    sw = table.reshape(V, D // 32, 2, 16).transpose(0, 1, 3, 2)
    bf = sw.reshape(V, D // 2, 2).astype(jnp.bfloat16)
    return jax.lax.bitcast_convert_type(bf, jnp.int32)


def kernel(move_tokens, move_colors, token_table, pos_table, color_table):
    B, T = move_tokens.shape
    D = token_table.shape[1]
    tok_idx = move_tokens.reshape(-1).astype(jnp.int32)
    cols = move_colors.astype(jnp.int32)
    th = jnp.arange(T // 2, dtype=jnp.int32)
    pc2_idx = ((cols[:, 0::2] * 3 + cols[:, 1::2]) * (T // 2)
               + th[None, :]).reshape(-1)
    pc2_table = _build_pc2_table(pos_table[:T], color_table).reshape(-1, 2 * D)
    out = _sc_gather_add(_swizzle_bf16(token_table), pc2_table,
                         tok_idx, pc2_idx)
    return out.reshape(B, T, D)


# speedup vs baseline: 1.7335x; 1.7335x over previous
import dataclasses
import functools

import jax
import jax.numpy as jnp
from jax import lax
from jax.experimental import pallas as pl
from jax.experimental.pallas import tpu as pltpu
from jax.experimental.pallas import tpu_sc as plsc

NC = 2
NS = 16
L = 16
NW = NC * NS


def _pc_body(pos_ref, col_ref, o_ref):
    o_ref[...] = pos_ref[...][None, :, :] + col_ref[...][:, None, :]


def _build_pc_table(pos_t, color_table):
    T, D = pos_t.shape
    C = color_table.shape[0]
    return pl.pallas_call(
        _pc_body,
        out_shape=jax.ShapeDtypeStruct((C, T, D), jnp.float32),
    )(pos_t, color_table)


def _sc_gather_add(tok_packed, pc_table, tok_idx, pc_idx, W=40, K=5, A=3,
                   TOK_SPLIT=(24, 16)):
    N = tok_idx.shape[0]
    Dw = tok_packed.shape[1]
    D = pc_table.shape[1]
    b_per_w = N // NW
    steps = b_per_w // W
    assert N % NW == 0 and b_per_w % W == 0 and sum(TOK_SPLIT) == W
    assert steps % K == 0 and steps >= 2 * K and A < K
    mesh = plsc.VectorSubcoreMesh(core_axis_name="c", subcore_axis_name="s")

    scratch = (
        [pltpu.VMEM((b_per_w,), jnp.int32)] * 2
        + [pltpu.VMEM((W, Dw), jnp.int32)] * K
        + [pltpu.VMEM((W, D), jnp.float32)] * K
        + [pltpu.SemaphoreType.DMA] * (2 * K)
    )

    cp = pltpu.CompilerParams()
    if "needs_layout_passes" in pltpu.CompilerParams.__dataclass_fields__:
        cp = dataclasses.replace(cp, needs_layout_passes=False)

    @functools.partial(
        pl.kernel,
        mesh=mesh,
        out_type=jax.ShapeDtypeStruct((N, D), jnp.float32),
        scratch_types=scratch,
        compiler_params=cp,
    )
    def k(tok_tab, pc_tab, tok_idx_hbm, pc_idx_hbm, out_hbm, *sc):
        tok_i_v, pc_i_v = sc[0], sc[1]
        ra = sc[2:2 + K]
        rb = sc[2 + K:2 + 2 * K]
        sg = sc[2 + 2 * K:2 + 3 * K]
        so = sc[2 + 3 * K:2 + 4 * K]
        wid = lax.axis_index("s") * NC + lax.axis_index("c")
        base_w = wid * b_per_w

        def _al(x):
            return x if isinstance(x, int) else pl.multiple_of(x, 8)

        pltpu.sync_copy(tok_idx_hbm.at[pl.ds(_al(base_w), b_per_w)], tok_i_v)
        pltpu.sync_copy(pc_idx_hbm.at[pl.ds(_al(base_w), b_per_w)], pc_i_v)

        def g_toks(i, p):
            cps = []
            r0 = 0
            for w in TOK_SPLIT:
                off = _al(i * W + r0)
                cps.append(pltpu.make_async_copy(
                    tok_tab.at[tok_i_v.at[pl.ds(off, w)]],
                    ra[p].at[pl.ds(r0, w)], sg[p]))
                r0 += w
            return cps

        def g_pc(i, p):
            off = _al(i * W)
            return pltpu.make_async_copy(
                pc_tab.at[pc_i_v.at[pl.ds(off, W)]], rb[p], sg[p])

        def out_cp(i, p):
            off = _al(base_w + i * W)
            return pltpu.make_async_copy(
                rb[p], out_hbm.at[pl.ds(off, W)], so[p])

        def issue(i, p):
            for c in g_toks(i, p):
                c.start()
            g_pc(i, p).start()

        def wait_g(i, p):
            for c in g_toks(i, p):
                c.wait()
            g_pc(i, p).wait()

        def adds(p):
            @pl.loop(0, W)
            def _row(r):
                for g in range(D // (2 * L)):
                    w = plsc.bitcast(ra[p][r, pl.ds(L * g, L)],
                                     jnp.bfloat16)
                    lo, hi = plsc.unpack(
                        w, format=plsc.PackFormat.INTERLEAVED)
                    plsc.addupdate(rb[p].at[r, pl.ds(2 * L * g, L)], lo)
                    plsc.addupdate(rb[p].at[r, pl.ds(2 * L * g + L, L)], hi)

        def consume(i, p):
            wait_g(i, p)
            adds(p)
            out_cp(i, p).start()

        for i in range(A):
            issue(i, i % K)
        for i in range(K - A):
            issue(i + A, (i + A) % K)
            consume(i, i % K)

        @pl.loop(0, (steps - K) // K)
        def _grp(j):
            for p in range(K):
                i = (K - A) + j * K + p
                cbuf = (K - A + p) % K
                ibuf = (K - A + p + A) % K
                out_cp(i + A - K, ibuf).wait()
                issue(i + A, ibuf)
                consume(i, cbuf)

        for i in range(steps - A, steps):
            consume(i, i % K)
        for i in range(steps - K, steps):
            out_cp(i, i % K).wait()

    return k(tok_packed, pc_table, tok_idx, pc_idx)


def _swizzle_bf16(table):
    V, D = table.shape
    sw = table.reshape(V, D // 32, 2, 16).transpose(0, 1, 3, 2)
    bf = sw.reshape(V, D // 2, 2).astype(jnp.bfloat16)
    return jax.lax.bitcast_convert_type(bf, jnp.int32)


def kernel(move_tokens, move_colors, token_table, pos_table, color_table):
    B, T = move_tokens.shape
    D = token_table.shape[1]
    tok_idx = move_tokens.reshape(-1).astype(jnp.int32)
    pos_ids = jnp.arange(T, dtype=jnp.int32)
    pc_idx = (move_colors.astype(jnp.int32) * T + pos_ids[None, :]).reshape(-1)
    pc_table = _build_pc_table(pos_table[:T], color_table).reshape(-1, D)
    out = _sc_gather_add(_swizzle_bf16(token_table), pc_table,
                         tok_idx, pc_idx)
    return out.reshape(B, T, D)


# speedup vs baseline: 1.7423x; 1.0051x over previous
import dataclasses
import functools

import jax
import jax.numpy as jnp
from jax import lax
from jax.experimental import pallas as pl
from jax.experimental.pallas import tpu as pltpu
from jax.experimental.pallas import tpu_sc as plsc

NC = 2
NS = 16
L = 16
NW = NC * NS


def _pc_body(pos_ref, col_ref, o_ref):
    o_ref[...] = pos_ref[...][None, :, :] + col_ref[...][:, None, :]


def _build_pc_table(pos_t, color_table):
    T, D = pos_t.shape
    C = color_table.shape[0]
    return pl.pallas_call(
        _pc_body,
        out_shape=jax.ShapeDtypeStruct((C, T, D), jnp.float32),
    )(pos_t, color_table)


def _sc_gather_add(tok_packed, pc_table, tok_idx, pc_idx, W=40, K=5, A=3,
                   TOK_SPLIT=(24, 16)):
    N = tok_idx.shape[0]
    Dw = tok_packed.shape[1]
    D = pc_table.shape[1]
    b_per_w = N // NW
    steps = b_per_w // W
    assert N % NW == 0 and b_per_w % W == 0 and sum(TOK_SPLIT) == W
    assert steps % K == 0 and steps >= 2 * K and A < K
    mesh = plsc.VectorSubcoreMesh(core_axis_name="c", subcore_axis_name="s")

    scratch = (
        [pltpu.VMEM((b_per_w,), jnp.int32)] * 2
        + [pltpu.VMEM((W, Dw), jnp.int32)] * K
        + [pltpu.VMEM((W, D), jnp.float32)] * K
        + [pltpu.SemaphoreType.DMA] * (2 * K)
    )

    cp = pltpu.CompilerParams()
    if "needs_layout_passes" in pltpu.CompilerParams.__dataclass_fields__:
        cp = dataclasses.replace(cp, needs_layout_passes=False)

    @functools.partial(
        pl.kernel,
        mesh=mesh,
        out_type=jax.ShapeDtypeStruct((N, D), jnp.float32),
        scratch_types=scratch,
        compiler_params=cp,
    )
    def k(tok_tab, pc_tab, tok_idx_hbm, pc_idx_hbm, out_hbm, *sc):
        tok_i_v, pc_i_v = sc[0], sc[1]
        ra = sc[2:2 + K]
        rb = sc[2 + K:2 + 2 * K]
        sg = sc[2 + 2 * K:2 + 3 * K]
        so = sc[2 + 3 * K:2 + 4 * K]
        wid = lax.axis_index("s") * NC + lax.axis_index("c")
        base_w = wid * b_per_w

        def _al(x):
            return x if isinstance(x, int) else pl.multiple_of(x, 8)

        pltpu.sync_copy(tok_idx_hbm.at[pl.ds(_al(base_w), b_per_w)], tok_i_v)
        pltpu.sync_copy(pc_idx_hbm.at[pl.ds(_al(base_w), b_per_w)], pc_i_v)

        def g_toks(i, p):
            cps = []
            r0 = 0
            for w in TOK_SPLIT:
                off = _al(i * W + r0)
                cps.append(pltpu.make_async_copy(
                    tok_tab.at[tok_i_v.at[pl.ds(off, w)]],
                    ra[p].at[pl.ds(r0, w)], sg[p]))
                r0 += w
            return cps

        def g_pc(i, p):
            off = _al(i * W)
            return pltpu.make_async_copy(
                pc_tab.at[pc_i_v.at[pl.ds(off, W)]], rb[p], sg[p])

        def out_cp(i, p):
            off = _al(base_w + i * W)
            return pltpu.make_async_copy(
                rb[p], out_hbm.at[pl.ds(off, W)], so[p])

        def issue(i, p):
            for c in g_toks(i, p):
                c.start()
            g_pc(i, p).start()

        def wait_g(i, p):
            for c in g_toks(i, p):
                c.wait()
            g_pc(i, p).wait()

        def adds(p):
            @pl.loop(0, W)
            def _row(r):
                for g in range(D // (2 * L)):
                    w = plsc.bitcast(ra[p][r, pl.ds(L * g, L)],
                                     jnp.bfloat16)
                    lo, hi = plsc.unpack(
                        w, format=plsc.PackFormat.INTERLEAVED)
                    plsc.addupdate(rb[p].at[r, pl.ds(2 * L * g, L)], lo)
                    plsc.addupdate(rb[p].at[r, pl.ds(2 * L * g + L, L)], hi)

        def consume(i, p):
            wait_g(i, p)
            adds(p)
            out_cp(i, p).start()

        for i in range(A):
            issue(i, i % K)
        for i in range(K - A):
            issue(i + A, (i + A) % K)
            consume(i, i % K)

        @pl.loop(0, (steps - K) // K)
        def _grp(j):
            for p in range(K):
                i = (K - A) + j * K + p
                cbuf = (K - A + p) % K
                ibuf = (K - A + p + A) % K
                out_cp(i + A - K, ibuf).wait()
                issue(i + A, ibuf)
                consume(i, cbuf)

        for i in range(steps - A, steps):
            consume(i, i % K)
        for i in range(steps - K, steps):
            out_cp(i, i % K).wait()

    return k(tok_packed, pc_table, tok_idx, pc_idx)


def _swizzle_bf16(table):
    V, D = table.shape
    sw = table.reshape(V, D // 32, 2, 16).transpose(0, 1, 3, 2)
    bf = sw.reshape(V, D // 2, 2).astype(jnp.bfloat16)
    return jax.lax.bitcast_convert_type(bf, jnp.int32)


def kernel(move_tokens, move_colors, token_table, pos_table, color_table):
    B, T = move_tokens.shape
    D = token_table.shape[1]
    tok_idx = move_tokens.reshape(-1).astype(jnp.int32)
    pos_ids = jnp.arange(T, dtype=jnp.int32)
    pc_idx = (move_colors.astype(jnp.int32) * T + pos_ids[None, :]).reshape(-1)
    pc_table = _build_pc_table(pos_table[:T], color_table).reshape(-1, D)
    out = _sc_gather_add(_swizzle_bf16(token_table), pc_table,
                         tok_idx, pc_idx, W=64, K=4, A=2, TOK_SPLIT=(64,))
    return out.reshape(B, T, D)
